# Initial kernel scaffold; baseline (speedup 1.0000x reference)
import jax
import jax.numpy as jnp
from jax.experimental import pallas as pl


def kernel(x, edge_index, Lsym, W_high, W_conv, b_conv, aL, aH):
    raise NotImplementedError("write your pallas kernel here")


# speedup vs baseline: 9.9275x; 9.9275x over previous
import functools

import jax
import jax.numpy as jnp
from jax import lax
from jax.experimental import pallas as pl
from jax.experimental.pallas import tpu as pltpu
from jax.experimental.pallas import tpu_sc as plsc

NC = 2
NS = 16
NW = NC * NS
LANES = 16
G = 128


def _sc_mesh():
    return plsc.VectorSubcoreMesh(
        core_axis_name="c", subcore_axis_name="s", num_cores=NC,
        num_subcores=NS)


def _zero_vmem_1d(ref, nwords):
    zeros = jnp.zeros((LANES,), jnp.float32)

    def body(i, _):
        ref[pl.ds(i * LANES, LANES)] = zeros
        return 0

    lax.fori_loop(0, nwords // LANES, body, 0)


def _make_deg_kernel(n_pad, ch):
    @functools.partial(
        pl.kernel,
        out_type=jax.ShapeDtypeStruct((NC, n_pad), jnp.float32),
        mesh=_sc_mesh(),
        scratch_types=[
            pltpu.VMEM((ch, G), jnp.int32),
            pltpu.VMEM((G,), jnp.float32),
            pltpu.VMEM((n_pad // NS,), jnp.float32),
            pltpu.VMEM_SHARED((n_pad,), jnp.float32),
        ],
    )
    def deg_kernel(dst_hbm, out_hbm, dst_v, ones_v, zstripe_v, sh_deg):
        cid = lax.axis_index("c")
        sid = lax.axis_index("s")
        wid = sid * NC + cid
        stripe = n_pad // NS

        _zero_vmem_1d(zstripe_v, stripe)
        pltpu.sync_copy(zstripe_v, sh_deg.at[pl.ds(sid * stripe, stripe)])

        def ones_body(i, _):
            ones_v[pl.ds(i * LANES, LANES)] = jnp.ones((LANES,), jnp.float32)
            return 0
        lax.fori_loop(0, G // LANES, ones_body, 0)

        pltpu.sync_copy(dst_hbm.at[wid], dst_v)
        plsc.subcore_barrier()

        def step(j, _):
            pltpu.sync_copy(ones_v, sh_deg.at[dst_v.at[j]], add=True)
            return 0
        lax.fori_loop(0, ch, step, 0)

        plsc.subcore_barrier()
        pltpu.sync_copy(sh_deg.at[pl.ds(sid * stripe, stripe)],
                        out_hbm.at[cid, pl.ds(sid * stripe, stripe)])

    return deg_kernel


def _make_scatter_kernel(n, n_pad, ch, d):
    @functools.partial(
        pl.kernel,
        out_type=jax.ShapeDtypeStruct((NC, n_pad, d), jnp.float32),
        mesh=_sc_mesh(),
        scratch_types=[
            pltpu.VMEM((ch, G), jnp.int32),
            pltpu.VMEM((ch, G), jnp.int32),
            pltpu.VMEM((G, d), jnp.float32),
            pltpu.VMEM((G, d), jnp.float32),
            pltpu.VMEM_SHARED((n_pad, d), jnp.float32),
            pltpu.SemaphoreType.DMA,
        ],
    )
    def scatter_kernel(src_hbm, dst_hbm, yw_hbm, out_hbm,
                       src_v, dst_v, rows_v, zero_v, sh_s, gsem):
        cid = lax.axis_index("c")
        sid = lax.axis_index("s")
        wid = sid * NC + cid
        stripe = n_pad // NS

        zeros = jnp.zeros((LANES,), jnp.float32)

        def zbody(i, _):
            r = i // (d // LANES)
            c = i % (d // LANES)
            zero_v[r, pl.ds(c * LANES, LANES)] = zeros
            return 0
        lax.fori_loop(0, G * d // LANES, zbody, 0)

        for k in range(stripe // G):
            pltpu.sync_copy(
                zero_v, sh_s.at[pl.ds(sid * stripe + k * G, G)])

        pltpu.sync_copy(src_hbm.at[wid], src_v)
        pltpu.sync_copy(dst_hbm.at[wid], dst_v)
        plsc.subcore_barrier()

        def step(j, _):
            pltpu.async_copy(yw_hbm.at[src_v.at[j]], rows_v, gsem).wait()
            pltpu.sync_copy(rows_v, sh_s.at[dst_v.at[j]], add=True)
            return 0
        lax.fori_loop(0, ch, step, 0)

        plsc.subcore_barrier()
        pltpu.sync_copy(sh_s.at[pl.ds(sid * stripe, stripe)],
                        out_hbm.at[cid, pl.ds(sid * stripe, stripe)])

    return scatter_kernel


def _prep_body(deg_ref, x_ref, wh_ref, wc_ref, al_ref, ah_ref,
               u_ref, yw_ref, dinv_ref):
    x = x_ref[...]
    u = lax.dot_general(x, wh_ref[...], (((1,), (1,)), ((), ())),
                        preferred_element_type=jnp.float32)
    u_ref[...] = ah_ref[0, 0] * jnp.maximum(u, 0.0)
    deg = deg_ref[:, 0:1] + deg_ref[:, 1:2] + 1.0
    dinv = lax.rsqrt(deg)
    dinv_ref[...] = dinv
    xw = lax.dot_general(x, wc_ref[...], (((1,), (1,)), ((), ())),
                         preferred_element_type=jnp.float32)
    yw_ref[...] = (al_ref[0, 0] * dinv) * xw


def _mm_body(lsym_ref, u_ref, s0_ref, s1_ref, yw_ref, dinv_ref, b_ref,
             al_ref, o_ref):
    acc = lax.dot_general(lsym_ref[...], u_ref[...], (((1,), (0,)), ((), ())),
                          preferred_element_type=jnp.float32)
    low = dinv_ref[...] * (s0_ref[...] + s1_ref[...] + yw_ref[...])
    o_ref[...] = acc + low + al_ref[0, 0] * b_ref[...]


def kernel(x, edge_index, Lsym, W_high, W_conv, b_conv, aL, aH):
    n, d_in = x.shape
    d = W_conv.shape[0]
    e = edge_index.shape[1]

    ew = -(-e // (NW * G)) * G
    ch = ew // G
    e_pad = ew * NW
    n_pad = -(-(n + 1) // (NS * G)) * (NS * G)

    src = edge_index[0]
    dst = edge_index[1]
    pad = e_pad - e
    src_p = jnp.concatenate([src, jnp.zeros((pad,), jnp.int32)])
    dst_p = jnp.concatenate([dst, jnp.full((pad,), n, jnp.int32)])
    src3 = src_p.reshape(NW, ch, G)
    dst3 = dst_p.reshape(NW, ch, G)

    deg2 = _make_deg_kernel(n_pad, ch)(dst3)
    degT = deg2[:, :n].T

    rb1 = 2000
    al2 = aL.reshape(1, 1)
    ah2 = aH.reshape(1, 1)
    u, yw, dinv = pl.pallas_call(
        _prep_body,
        grid=(n // rb1,),
        in_specs=[
            pl.BlockSpec((rb1, NC), lambda i: (i, 0)),
            pl.BlockSpec((rb1, d_in), lambda i: (i, 0)),
            pl.BlockSpec((d, d_in), lambda i: (0, 0)),
            pl.BlockSpec((d, d_in), lambda i: (0, 0)),
            pl.BlockSpec(memory_space=pltpu.SMEM),
            pl.BlockSpec(memory_space=pltpu.SMEM),
        ],
        out_specs=[
            pl.BlockSpec((rb1, d), lambda i: (i, 0)),
            pl.BlockSpec((rb1, d), lambda i: (i, 0)),
            pl.BlockSpec((rb1, 1), lambda i: (i, 0)),
        ],
        out_shape=[
            jax.ShapeDtypeStruct((n, d), jnp.float32),
            jax.ShapeDtypeStruct((n, d), jnp.float32),
            jax.ShapeDtypeStruct((n, 1), jnp.float32),
        ],
    )(degT, x, W_high, W_conv, al2, ah2)

    s2 = _make_scatter_kernel(n, n_pad, ch, d)(src3, dst3, yw)
    s0 = s2[0, :n]
    s1 = s2[1, :n]

    rb2 = 400
    out = pl.pallas_call(
        _mm_body,
        grid=(n // rb2,),
        in_specs=[
            pl.BlockSpec((rb2, n), lambda i: (i, 0)),
            pl.BlockSpec((n, d), lambda i: (0, 0)),
            pl.BlockSpec((rb2, d), lambda i: (i, 0)),
            pl.BlockSpec((rb2, d), lambda i: (i, 0)),
            pl.BlockSpec((rb2, d), lambda i: (i, 0)),
            pl.BlockSpec((rb2, 1), lambda i: (i, 0)),
            pl.BlockSpec((1, d), lambda i: (0, 0)),
            pl.BlockSpec(memory_space=pltpu.SMEM),
        ],
        out_specs=pl.BlockSpec((rb2, d), lambda i: (i, 0)),
        out_shape=jax.ShapeDtypeStruct((n, d), jnp.float32),
    )(Lsym, u, s0, s1, yw, dinv, b_conv.reshape(1, d), al2)
    return out
